# baseline (speedup 1.0000x reference)
import functools

import jax
import jax.numpy as jnp
from jax.experimental import pallas as pl

_B, _S, _D = 4, 8192, 128
_E, _K = 64, 8
_ALPHA = 0.01
_T = _B * _S
_TB = 8192
_NBLK = _T // _TB
_SBLK = _S // _TB


def _gate_kernel(x_ref, w_ref, b_ref, probs_ref, idx_ref, wgt_ref,
                 freq_ref, aux_ref):
    i = pl.program_id(0)

    x = x_ref[0]
    w = w_ref[...]
    scores = jax.lax.dot_general(
        w, x, (((1,), (1,)), ((), ())),
        preferred_element_type=jnp.float32) + b_ref[...]

    e = jnp.exp(scores)
    p = e / jnp.sum(e, axis=0, keepdims=True)
    probs_ref[0] = p.T

    sub = jax.lax.broadcasted_iota(jnp.int32, p.shape, 0)
    pb = jax.lax.bitcast_convert_type(p, jnp.int32)
    enc = (pb & jnp.int32(-64)) | (jnp.int32(_E - 1) - sub)
    work = jax.lax.bitcast_convert_type(enc, jnp.float32)

    rows = []
    for _ in range(_K):
        mval = jnp.max(work, axis=0, keepdims=True)
        rows.append(mval)
        work = jnp.where(work == mval, -jnp.inf, work)
    mb = jax.lax.bitcast_convert_type(
        jnp.concatenate(rows, axis=0), jnp.int32)
    idx_ref[0] = (jnp.int32(_E - 1) - (mb & jnp.int32(_E - 1))).T
    wgt_ref[0] = jax.lax.bitcast_convert_type(
        mb & jnp.int32(-64), jnp.float32).T

    masked = jnp.where(work < 0.0, p, 0.0)
    block_sum = jnp.sum(masked, axis=1, keepdims=True)

    @pl.when(i == 0)
    def _init():
        freq_ref[...] = jnp.zeros_like(freq_ref)

    freq_ref[...] += block_sum

    @pl.when(i == _NBLK - 1)
    def _fin():
        freq = freq_ref[...] * (1.0 / _T)
        aux = jnp.mean((freq - 1.0 / _E) ** 2) * _ALPHA
        aux_ref[...] = aux.reshape(1, 1)


@functools.partial(jax.jit, static_argnames=("interpret",))
def kernel(x, W, b, interpret=False):
    b2 = b.reshape(_E, 1)
    probs, idx, wgt, _freq, aux = pl.pallas_call(
        _gate_kernel,
        grid=(_NBLK,),
        in_specs=[
            pl.BlockSpec((1, _TB, _D), lambda i: (i // _SBLK, i % _SBLK, 0)),
            pl.BlockSpec((_E, _D), lambda i: (0, 0)),
            pl.BlockSpec((_E, 1), lambda i: (0, 0)),
        ],
        out_specs=[
            pl.BlockSpec((1, _TB, _E), lambda i: (i // _SBLK, i % _SBLK, 0)),
            pl.BlockSpec((1, _TB, _K), lambda i: (i // _SBLK, i % _SBLK, 0)),
            pl.BlockSpec((1, _TB, _K), lambda i: (i // _SBLK, i % _SBLK, 0)),
            pl.BlockSpec((_E, 1), lambda i: (0, 0)),
            pl.BlockSpec((1, 1), lambda i: (0, 0)),
        ],
        out_shape=[
            jax.ShapeDtypeStruct((_B, _S, _E), jnp.float32),
            jax.ShapeDtypeStruct((_B, _S, _K), jnp.int32),
            jax.ShapeDtypeStruct((_B, _S, _K), jnp.float32),
            jax.ShapeDtypeStruct((_E, 1), jnp.float32),
            jax.ShapeDtypeStruct((1, 1), jnp.float32),
        ],
        interpret=interpret,
    )(x, W, b2)
    return (idx, wgt, aux.reshape(()), probs)


# speedup vs baseline: 1.8071x; 1.8071x over previous
import functools

import jax
import jax.numpy as jnp
from jax.experimental import pallas as pl

_B, _S, _D = 4, 8192, 128
_E, _K = 64, 8
_ALPHA = 0.01
_T = _B * _S
_TB = 4096
_NBLK = _T // _TB
_SBLK = _S // _TB


def _gate_kernel(x_ref, w_ref, b_ref, probs_ref, idx_ref, wgt_ref,
                 freq_ref, aux_ref):
    i = pl.program_id(0)

    x = x_ref[0]
    w = w_ref[...]
    scores = jax.lax.dot_general(
        w, x, (((1,), (1,)), ((), ())),
        preferred_element_type=jnp.float32) + b_ref[...]

    e = jnp.exp(scores)
    p = e / jnp.sum(e, axis=0, keepdims=True)
    probs_ref[0] = p.T

    sub = jax.lax.broadcasted_iota(jnp.int32, p.shape, 0)
    pb = jax.lax.bitcast_convert_type(p, jnp.int32)
    enc = (pb & jnp.int32(-64)) | (jnp.int32(_E - 1) - sub)
    work = jax.lax.bitcast_convert_type(enc, jnp.float32)

    rows = []
    for _ in range(_K):
        mval = jnp.max(work, axis=0, keepdims=True)
        rows.append(mval)
        work = jnp.where(work == mval, -jnp.inf, work)
    mb = jax.lax.bitcast_convert_type(
        jnp.concatenate(rows, axis=0), jnp.int32)
    idx_ref[0] = jnp.int32(_E - 1) - (mb & jnp.int32(_E - 1))
    wgt_ref[0] = jax.lax.bitcast_convert_type(
        mb & jnp.int32(-64), jnp.float32)

    masked = jnp.where(work < 0.0, p, 0.0)
    block_sum = jnp.sum(masked, axis=1, keepdims=True)

    @pl.when(i == 0)
    def _init():
        freq_ref[...] = jnp.zeros_like(freq_ref)

    freq_ref[...] += block_sum

    @pl.when(i == _NBLK - 1)
    def _fin():
        freq = freq_ref[...] * (1.0 / _T)
        aux = jnp.mean((freq - 1.0 / _E) ** 2) * _ALPHA
        aux_ref[...] = aux.reshape(1, 1)


@functools.partial(jax.jit, static_argnames=("interpret",))
def kernel(x, W, b, interpret=False):
    b2 = b.reshape(_E, 1)
    probs, idx, wgt, _freq, aux = pl.pallas_call(
        _gate_kernel,
        grid=(_NBLK,),
        in_specs=[
            pl.BlockSpec((1, _TB, _D), lambda i: (i // _SBLK, i % _SBLK, 0)),
            pl.BlockSpec((_E, _D), lambda i: (0, 0)),
            pl.BlockSpec((_E, 1), lambda i: (0, 0)),
        ],
        out_specs=[
            pl.BlockSpec((1, _TB, _E), lambda i: (i // _SBLK, i % _SBLK, 0)),
            pl.BlockSpec((1, _K, _TB), lambda i: (i // _SBLK, 0, i % _SBLK)),
            pl.BlockSpec((1, _K, _TB), lambda i: (i // _SBLK, 0, i % _SBLK)),
            pl.BlockSpec((_E, 1), lambda i: (0, 0)),
            pl.BlockSpec((1, 1), lambda i: (0, 0)),
        ],
        out_shape=[
            jax.ShapeDtypeStruct((_B, _S, _E), jnp.float32),
            jax.ShapeDtypeStruct((_B, _K, _S), jnp.int32),
            jax.ShapeDtypeStruct((_B, _K, _S), jnp.float32),
            jax.ShapeDtypeStruct((_E, 1), jnp.float32),
            jax.ShapeDtypeStruct((1, 1), jnp.float32),
        ],
        interpret=interpret,
    )(x, W, b2)
    return (jnp.transpose(idx, (0, 2, 1)), jnp.transpose(wgt, (0, 2, 1)),
            aux.reshape(()), probs)


# speedup vs baseline: 1.8681x; 1.0338x over previous
import functools

import jax
import jax.numpy as jnp
from jax.experimental import pallas as pl

_B, _S, _D = 4, 8192, 128
_E, _K = 64, 8
_ALPHA = 0.01
_T = _B * _S
_TB = 8192
_NBLK = _T // _TB
_SBLK = _S // _TB


def _gate_kernel(x_ref, w_ref, b_ref, probs_ref, idx_ref, wgt_ref,
                 freq_ref, aux_ref):
    i = pl.program_id(0)

    x = x_ref[0]
    w = w_ref[...]
    scores = jax.lax.dot_general(
        w, x, (((1,), (1,)), ((), ())),
        preferred_element_type=jnp.float32) + b_ref[...]

    e = jnp.exp(scores)
    p = e / jnp.sum(e, axis=0, keepdims=True)
    probs_ref[0] = p.T

    sub = jax.lax.broadcasted_iota(jnp.int32, p.shape, 0)
    pb = jax.lax.bitcast_convert_type(p, jnp.int32)
    enc = (pb & jnp.int32(-64)) | (jnp.int32(_E - 1) - sub)
    work = jax.lax.bitcast_convert_type(enc, jnp.float32)

    rows = []
    for _ in range(_K):
        mval = jnp.max(work, axis=0, keepdims=True)
        rows.append(mval)
        work = jnp.where(work == mval, -jnp.inf, work)
    mb = jax.lax.bitcast_convert_type(
        jnp.concatenate(rows, axis=0), jnp.int32)
    idx_ref[0] = jnp.int32(_E - 1) - (mb & jnp.int32(_E - 1))
    wgt_ref[0] = jax.lax.bitcast_convert_type(
        mb & jnp.int32(-64), jnp.float32)

    masked = jnp.where(work < 0.0, p, 0.0)
    block_sum = jnp.sum(masked, axis=1, keepdims=True)

    @pl.when(i == 0)
    def _init():
        freq_ref[...] = jnp.zeros_like(freq_ref)

    freq_ref[...] += block_sum

    @pl.when(i == _NBLK - 1)
    def _fin():
        freq = freq_ref[...] * (1.0 / _T)
        aux = jnp.mean((freq - 1.0 / _E) ** 2) * _ALPHA
        aux_ref[...] = aux.reshape(1, 1)


@functools.partial(jax.jit, static_argnames=("interpret",))
def kernel(x, W, b, interpret=False):
    b2 = b.reshape(_E, 1)
    probs, idx, wgt, _freq, aux = pl.pallas_call(
        _gate_kernel,
        grid=(_NBLK,),
        in_specs=[
            pl.BlockSpec((1, _TB, _D), lambda i: (i // _SBLK, i % _SBLK, 0)),
            pl.BlockSpec((_E, _D), lambda i: (0, 0)),
            pl.BlockSpec((_E, 1), lambda i: (0, 0)),
        ],
        out_specs=[
            pl.BlockSpec((1, _TB, _E), lambda i: (i // _SBLK, i % _SBLK, 0)),
            pl.BlockSpec((1, _K, _TB), lambda i: (i // _SBLK, 0, i % _SBLK)),
            pl.BlockSpec((1, _K, _TB), lambda i: (i // _SBLK, 0, i % _SBLK)),
            pl.BlockSpec((_E, 1), lambda i: (0, 0)),
            pl.BlockSpec((1, 1), lambda i: (0, 0)),
        ],
        out_shape=[
            jax.ShapeDtypeStruct((_B, _S, _E), jnp.float32),
            jax.ShapeDtypeStruct((_B, _K, _S), jnp.int32),
            jax.ShapeDtypeStruct((_B, _K, _S), jnp.float32),
            jax.ShapeDtypeStruct((_E, 1), jnp.float32),
            jax.ShapeDtypeStruct((1, 1), jnp.float32),
        ],
        interpret=interpret,
    )(x, W, b2)
    return (jnp.transpose(idx, (0, 2, 1)), jnp.transpose(wgt, (0, 2, 1)),
            aux.reshape(()), probs)
